# baseline (speedup 1.0000x reference)
import jax
import jax.numpy as jnp
from jax.experimental import pallas as pl
from jax.experimental.pallas import tpu as pltpu

_N = 5000
_B = 256
_R = 20
_NP = _R * _B
_T = 0.5


def _nms_body(x1r, y1r, x2r, y2r, ar, sr,
              x1t, y1t, x2t, y2t, at,
              out,
              sup):
    sup[...] = jnp.zeros((_R, _B), jnp.float32)

    def block_step(r, _):
        oh = (jax.lax.broadcasted_iota(jnp.int32, (_B, _R), 1) == r).astype(
            jnp.float32)
        cx1 = jnp.sum(x1t[...] * oh, axis=1, keepdims=True)
        cy1 = jnp.sum(y1t[...] * oh, axis=1, keepdims=True)
        cx2 = jnp.sum(x2t[...] * oh, axis=1, keepdims=True)
        cy2 = jnp.sum(y2t[...] * oh, axis=1, keepdims=True)
        ca = jnp.sum(at[...] * oh, axis=1, keepdims=True)

        def ov_tile(rr):
            jx1 = x1r[pl.ds(rr, 1), :]
            jy1 = y1r[pl.ds(rr, 1), :]
            jx2 = x2r[pl.ds(rr, 1), :]
            jy2 = y2r[pl.ds(rr, 1), :]
            ja = ar[pl.ds(rr, 1), :]
            w = jnp.maximum(0.0, jnp.minimum(cx2, jx2) - jnp.maximum(cx1, jx1))
            h = jnp.maximum(0.0, jnp.minimum(cy2, jy2) - jnp.maximum(cy1, jy1))
            inter = w * h
            iou = inter / (ca + ja - inter)
            return (iou > _T).astype(jnp.float32)

        rowi = jax.lax.broadcasted_iota(jnp.int32, (_B, _B), 0)
        coli = jax.lax.broadcasted_iota(jnp.int32, (_B, _B), 1)
        pf = ov_tile(r) * (coli > rowi).astype(jnp.float32)
        notsup0 = 1.0 - sup[pl.ds(r, 1), :]

        def jac_cond(state):
            return state[1]

        def jac_body(state):
            k, _ = state
            supped = (jnp.dot(k, pf, preferred_element_type=jnp.float32)
                      > 0.5).astype(jnp.float32)
            knew = notsup0 * (1.0 - supped)
            return knew, jnp.any(knew != k)

        keepb, _ = jax.lax.while_loop(jac_cond, jac_body, (notsup0, True))
        out[pl.ds(r, 1), :] = sr[pl.ds(r, 1), :] * keepb

        def later8(t, _):
            base = r + 1 + 2 * t
            for u in range(2):
                rr = jnp.minimum(base + u, _R - 1)
                ovf = ov_tile(rr)
                supadd = jnp.dot(keepb, ovf,
                                 preferred_element_type=jnp.float32)
                srow = sup[pl.ds(rr, 1), :]
                sup[pl.ds(rr, 1), :] = jnp.maximum(
                    srow, (supadd > 0.5).astype(jnp.float32))
            return 0

        nchunks = (_R - 1 - r + 1) // 2
        jax.lax.fori_loop(0, nchunks, later8, 0)
        return 0

    jax.lax.fori_loop(0, _R, block_step, 0)


def kernel(boxes, scores):
    area = (boxes[:, 2] - boxes[:, 0]) * (boxes[:, 3] - boxes[:, 1])
    srt = jax.lax.sort((-scores, boxes[:, 0], boxes[:, 1], boxes[:, 2],
                        boxes[:, 3], area, scores),
                       dimension=0, is_stable=True, num_keys=1)
    g = jnp.stack(srt[1:], axis=1)
    pad_rows = jnp.tile(jnp.array([[-1e6, -1e6, -1e6 + 1.0, -1e6 + 1.0,
                                    1.0, 0.0]], dtype=jnp.float32),
                        (_NP - _N, 1))
    gp = jnp.concatenate([g, pad_rows], axis=0)
    cols = [gp[:, i].reshape(_R, _B) for i in range(6)]
    args = tuple(cols) + tuple(c.T for c in cols[:5])
    out = pl.pallas_call(
        _nms_body,
        out_shape=jax.ShapeDtypeStruct((_R, _B), jnp.float32),
        scratch_shapes=[pltpu.VMEM((_R, _B), jnp.float32)],
    )(*args)
    return out.reshape(_NP)[:_N]


# speedup vs baseline: 1.1764x; 1.1764x over previous
import jax
import jax.numpy as jnp
from jax.experimental import pallas as pl
from jax.experimental.pallas import tpu as pltpu

_N = 5000
_B = 512
_R = 10
_NP = _R * _B
_T = 0.5


def _nms_body(x1r, y1r, x2r, y2r, ar, sr,
              x1t, y1t, x2t, y2t, at,
              out,
              sup):
    sup[...] = jnp.zeros((_R, _B), jnp.float32)

    def block_step(r, _):
        oh = (jax.lax.broadcasted_iota(jnp.int32, (_B, _R), 1) == r).astype(
            jnp.float32)
        cx1 = jnp.sum(x1t[...] * oh, axis=1, keepdims=True)
        cy1 = jnp.sum(y1t[...] * oh, axis=1, keepdims=True)
        cx2 = jnp.sum(x2t[...] * oh, axis=1, keepdims=True)
        cy2 = jnp.sum(y2t[...] * oh, axis=1, keepdims=True)
        ca = jnp.sum(at[...] * oh, axis=1, keepdims=True)

        def ov_tile(rr):
            jx1 = x1r[pl.ds(rr, 1), :]
            jy1 = y1r[pl.ds(rr, 1), :]
            jx2 = x2r[pl.ds(rr, 1), :]
            jy2 = y2r[pl.ds(rr, 1), :]
            ja = ar[pl.ds(rr, 1), :]
            w = jnp.maximum(0.0, jnp.minimum(cx2, jx2) - jnp.maximum(cx1, jx1))
            h = jnp.maximum(0.0, jnp.minimum(cy2, jy2) - jnp.maximum(cy1, jy1))
            inter = w * h
            iou = inter / (ca + ja - inter)
            return (iou > _T).astype(jnp.float32)

        rowi = jax.lax.broadcasted_iota(jnp.int32, (_B, _B), 0)
        coli = jax.lax.broadcasted_iota(jnp.int32, (_B, _B), 1)
        pf = ov_tile(r) * (coli > rowi).astype(jnp.float32)
        notsup0 = 1.0 - sup[pl.ds(r, 1), :]

        def jac_cond(state):
            return state[1]

        def jac_body(state):
            k, _ = state
            supped = (jnp.dot(k, pf, preferred_element_type=jnp.float32)
                      > 0.5).astype(jnp.float32)
            knew = notsup0 * (1.0 - supped)
            return knew, jnp.any(knew != k)

        keepb, _ = jax.lax.while_loop(jac_cond, jac_body, (notsup0, True))
        out[pl.ds(r, 1), :] = sr[pl.ds(r, 1), :] * keepb

        def later8(t, _):
            base = r + 1 + 2 * t
            for u in range(2):
                rr = jnp.minimum(base + u, _R - 1)
                ovf = ov_tile(rr)
                supadd = jnp.dot(keepb, ovf,
                                 preferred_element_type=jnp.float32)
                srow = sup[pl.ds(rr, 1), :]
                sup[pl.ds(rr, 1), :] = jnp.maximum(
                    srow, (supadd > 0.5).astype(jnp.float32))
            return 0

        nchunks = (_R - 1 - r + 1) // 2
        jax.lax.fori_loop(0, nchunks, later8, 0)
        return 0

    jax.lax.fori_loop(0, _R, block_step, 0)


def kernel(boxes, scores):
    area = (boxes[:, 2] - boxes[:, 0]) * (boxes[:, 3] - boxes[:, 1])
    srt = jax.lax.sort((-scores, boxes[:, 0], boxes[:, 1], boxes[:, 2],
                        boxes[:, 3], area, scores),
                       dimension=0, is_stable=True, num_keys=1)
    g = jnp.stack(srt[1:], axis=1)
    pad_rows = jnp.tile(jnp.array([[-1e6, -1e6, -1e6 + 1.0, -1e6 + 1.0,
                                    1.0, 0.0]], dtype=jnp.float32),
                        (_NP - _N, 1))
    gp = jnp.concatenate([g, pad_rows], axis=0)
    cols = [gp[:, i].reshape(_R, _B) for i in range(6)]
    args = tuple(cols) + tuple(c.T for c in cols[:5])
    out = pl.pallas_call(
        _nms_body,
        out_shape=jax.ShapeDtypeStruct((_R, _B), jnp.float32),
        scratch_shapes=[pltpu.VMEM((_R, _B), jnp.float32)],
    )(*args)
    return out.reshape(_NP)[:_N]
